# Initial kernel scaffold; baseline (speedup 1.0000x reference)
import functools

import jax
import jax.numpy as jnp
from jax import lax
from jax.experimental import pallas as pl
from jax.experimental.pallas import tpu as pltpu
from jax.experimental.pallas import tpu_sc as plsc

NUM_ITEMS = 1000000
LOOKBACK = 200
EMB_SIZE = 64
BATCH = 4096
SEQ_LEN = 200

NC = 2
NS = 16
NW = NC * NS
LANES = 16
VPR = EMB_SIZE // LANES

TOTAL_ROWS = BATCH * SEQ_LEN
ROWS_W = TOTAL_ROWS // NW
SEQS_W = ROWS_W // SEQ_LEN


def _sc_body(idx_hbm, item_hbm, pos_hbm, out_hbm, idx_v, pos_v, rows_v, gsem):
    wid = lax.axis_index("s") * NC + lax.axis_index("c")
    base = wid * ROWS_W

    pltpu.sync_copy(idx_hbm.at[pl.ds(base, ROWS_W)], idx_v)
    pltpu.sync_copy(pos_hbm, pos_v)

    @pl.loop(0, SEQS_W)
    def _seq(g):
        start = g * SEQ_LEN
        pltpu.async_copy(
            item_hbm.at[idx_v.at[pl.ds(start, SEQ_LEN)]], rows_v, gsem
        ).wait()

        @pl.loop(0, SEQ_LEN)
        def _row(l):
            for j in range(VPR):
                sl = pl.ds(j * LANES, LANES)
                rows_v[l, sl] = rows_v[l, sl] + pos_v[l, sl]

        pltpu.sync_copy(rows_v, out_hbm.at[pl.ds(base + start, SEQ_LEN)])


@jax.jit
def _run(idx_flat, item_table, pos_table):
    mesh = plsc.VectorSubcoreMesh(core_axis_name="c", subcore_axis_name="s")
    k = pl.kernel(
        _sc_body,
        out_type=jax.ShapeDtypeStruct((TOTAL_ROWS, EMB_SIZE), jnp.float32),
        mesh=mesh,
        scratch_types=[
            pltpu.VMEM((ROWS_W,), jnp.int32),
            pltpu.VMEM((LOOKBACK, EMB_SIZE), jnp.float32),
            pltpu.VMEM((SEQ_LEN, EMB_SIZE), jnp.float32),
            pltpu.SemaphoreType.DMA,
        ],
    )
    return k(idx_flat, item_table, pos_table)


def kernel(input_seqs, item_table, pos_table):
    idx_flat = input_seqs.reshape(-1).astype(jnp.int32)
    out = _run(idx_flat, item_table, pos_table)
    return out.reshape(BATCH, SEQ_LEN, EMB_SIZE)


# speedup vs baseline: 2.4579x; 2.4579x over previous
import functools

import jax
import jax.numpy as jnp
from jax import lax
from jax.experimental import pallas as pl
from jax.experimental.pallas import tpu as pltpu
from jax.experimental.pallas import tpu_sc as plsc

NUM_ITEMS = 1000000
LOOKBACK = 200
EMB_SIZE = 64
BATCH = 4096
SEQ_LEN = 200

NC = 2
NS = 16
NW = NC * NS
LANES = 16
VPR = EMB_SIZE // LANES

TOTAL_ROWS = BATCH * SEQ_LEN
ROWS_W = TOTAL_ROWS // NW
SEQS_W = ROWS_W // SEQ_LEN


def _sc_body(idx_hbm, item_hbm, pos_hbm, out_hbm, idx_v, pos_v, rows_v, gsem):
    wid = lax.axis_index("s") * NC + lax.axis_index("c")
    base = wid * ROWS_W

    pltpu.sync_copy(idx_hbm.at[pl.ds(base, ROWS_W)], idx_v)
    pltpu.sync_copy(pos_hbm, pos_v)

    @pl.loop(0, SEQS_W)
    def _seq(g):
        start = g * SEQ_LEN
        pltpu.async_copy(
            item_hbm.at[idx_v.at[pl.ds(start, SEQ_LEN)]], rows_v, gsem
        ).wait()

        @pl.loop(0, SEQ_LEN)
        def _row(l):
            for j in range(VPR):
                sl = pl.ds(j * LANES, LANES)
                rows_v[l, sl] = rows_v[l, sl] + pos_v[l, sl]

        pltpu.sync_copy(rows_v, out_hbm.at[pl.ds(base + start, SEQ_LEN)])


@jax.jit
def _run(idx_flat, item_table, pos_table):
    mesh = plsc.VectorSubcoreMesh(core_axis_name="c", subcore_axis_name="s")
    k = pl.kernel(
        _sc_body,
        out_type=jax.ShapeDtypeStruct((TOTAL_ROWS, EMB_SIZE), jnp.float32),
        mesh=mesh,
        scratch_types=[
            pltpu.VMEM((ROWS_W,), jnp.int32),
            pltpu.VMEM((LOOKBACK, EMB_SIZE), jnp.float32),
            pltpu.VMEM((SEQ_LEN, EMB_SIZE), jnp.float32),
            pltpu.SemaphoreType.DMA,
        ],
        compiler_params=pltpu.CompilerParams(use_tc_tiling_on_sc=False),
    )
    return k(idx_flat, item_table, pos_table)


def kernel(input_seqs, item_table, pos_table):
    idx_flat = input_seqs.reshape(-1).astype(jnp.int32)
    out = _run(idx_flat, item_table, pos_table)
    return out.reshape(BATCH, SEQ_LEN, EMB_SIZE)


# speedup vs baseline: 2.5490x; 1.0371x over previous
import jax
import jax.numpy as jnp
from jax import lax
from jax.experimental import pallas as pl
from jax.experimental.pallas import tpu as pltpu
from jax.experimental.pallas import tpu_sc as plsc

NUM_ITEMS = 1000000
LOOKBACK = 200
EMB_SIZE = 64
BATCH = 4096
SEQ_LEN = 200

NC = 2
NS = 16
NW = NC * NS
LANES = 16
VPR = EMB_SIZE // LANES

TOTAL_ROWS = BATCH * SEQ_LEN
ROWS_W = TOTAL_ROWS // NW
SEQS_W = ROWS_W // SEQ_LEN
NBUF = 4
AHEAD = 2


def _sc_body(idx_hbm, item_hbm, pos_hbm, out_hbm, idx_v, pos_v, rows_v,
             gsem, osem):
    wid = lax.axis_index("s") * NC + lax.axis_index("c")
    base = wid * ROWS_W

    pltpu.sync_copy(idx_hbm.at[pl.ds(base, ROWS_W)], idx_v)
    pltpu.sync_copy(pos_hbm, pos_v)

    def issue_gather(g, q):
        pltpu.async_copy(
            item_hbm.at[idx_v.at[pl.ds(g * SEQ_LEN, SEQ_LEN)]],
            rows_v.at[q],
            gsem.at[q],
        )

    for b in range(AHEAD):
        issue_gather(b, b)

    @pl.loop(0, SEQS_W, step=NBUF)
    def _group(go):
        for b in range(NBUF):
            g = go + b
            q = (b + AHEAD) % NBUF

            @pl.when(g + AHEAD < SEQS_W)
            def _issue():
                @pl.when(g >= NBUF - AHEAD)
                def _drain():
                    pltpu.make_async_copy(
                        rows_v.at[q],
                        out_hbm.at[pl.ds(0, SEQ_LEN)],
                        osem.at[q],
                    ).wait()

                issue_gather(g + AHEAD, q)

            pltpu.make_async_copy(
                item_hbm.at[idx_v.at[pl.ds(0, SEQ_LEN)]],
                rows_v.at[b],
                gsem.at[b],
            ).wait()

            @pl.loop(0, SEQ_LEN, unroll=8)
            def _row(l):
                for j in range(VPR):
                    sl = pl.ds(j * LANES, LANES)
                    rows_v[b, l, sl] = rows_v[b, l, sl] + pos_v[l, sl]

            pltpu.async_copy(
                rows_v.at[b],
                out_hbm.at[pl.ds(base + g * SEQ_LEN, SEQ_LEN)],
                osem.at[b],
            )

    for b in range(NBUF):
        pltpu.make_async_copy(
            rows_v.at[b],
            out_hbm.at[pl.ds(0, SEQ_LEN)],
            osem.at[b],
        ).wait()


@jax.jit
def _run(idx_flat, item_table, pos_table):
    mesh = plsc.VectorSubcoreMesh(core_axis_name="c", subcore_axis_name="s")
    k = pl.kernel(
        _sc_body,
        out_type=jax.ShapeDtypeStruct((TOTAL_ROWS, EMB_SIZE), jnp.float32),
        mesh=mesh,
        scratch_types=[
            pltpu.VMEM((ROWS_W,), jnp.int32),
            pltpu.VMEM((LOOKBACK, EMB_SIZE), jnp.float32),
            pltpu.VMEM((NBUF, SEQ_LEN, EMB_SIZE), jnp.float32),
            pltpu.SemaphoreType.DMA((NBUF,)),
            pltpu.SemaphoreType.DMA((NBUF,)),
        ],
        compiler_params=pltpu.CompilerParams(use_tc_tiling_on_sc=False),
    )
    return k(idx_flat, item_table, pos_table)


def kernel(input_seqs, item_table, pos_table):
    idx_flat = input_seqs.reshape(-1).astype(jnp.int32)
    out = _run(idx_flat, item_table, pos_table)
    return out.reshape(BATCH, SEQ_LEN, EMB_SIZE)
